# baseline (speedup 1.0000x reference)
import jax
import jax.numpy as jnp
from jax import lax
from jax.experimental import pallas as pl
from jax.experimental.pallas import tpu as pltpu
from jax.experimental.pallas import tpu_sc as plsc


def _sc_kernel(seq_rows, out_rows):
    w = lax.axis_index("s") * 2 + lax.axis_index("c")
    base = w * 12800
    pltpu.sync_copy(seq_rows.at[pl.ds(base, 12800)],
                    out_rows.at[pl.ds(base, 12800)])


@jax.jit
def kernel(seq, skill, obj_idx, obj_embed):
    seq_rows = seq.reshape(409600, 128)
    mesh = plsc.VectorSubcoreMesh(core_axis_name="c", subcore_axis_name="s")
    out4 = pl.kernel(
        _sc_kernel,
        out_type=jax.ShapeDtypeStruct((413696, 128), jnp.float32),
        mesh=mesh,
        compiler_params=pltpu.CompilerParams(use_tc_tiling_on_sc=False),
        scratch_types=[],
    )(seq_rows)
    out = (out4.reshape(512, 101, 8, 128).transpose(0, 2, 1, 3)
           .reshape(4096, 12928))
    return out


# speedup vs baseline: 19.6042x; 19.6042x over previous
import jax
import jax.numpy as jnp
from jax import lax
from jax.experimental import pallas as pl
from jax.experimental.pallas import tpu as pltpu
from jax.experimental.pallas import tpu_sc as plsc

OBS = 64
SEQ_LEN = 200
BATCH = 4096
OUT_COLS = SEQ_LEN * OBS + 128
L = 16
OCH = 40
CCH = OCH * OBS
NCHUNK = 80


def _sc_kernel(seq, skill, obj_idx, obj_embed, out,
               idxv, embv, skbuf, buf0, buf1,
               semi0, semi1, semo0, semo1, sems):
    w = lax.axis_index("s") * 2 + lax.axis_index("c")
    bufs = [buf0, buf1]
    sem_in = [semi0, semi1]
    sem_out = [semo0, semo1]

    pltpu.sync_copy(obj_idx, idxv.at[pl.ds(0, BATCH)])
    pltpu.sync_copy(obj_embed, embv)
    evecs = [embv[pl.ds(L * j, L)] for j in range(OBS // L)]

    b0 = pl.multiple_of(w * 128, 128)
    skill_in = pltpu.make_async_copy(
        skill.at[pl.ds(b0, 128), :], skbuf, sems)
    skill_in.start()

    def rg(t):
        r_ = pl.multiple_of((w * 16 + t // 5) * 8, 8)
        g_ = t % 5
        return r_, g_

    def start_in(kb, t):
        r_, g_ = rg(t)
        pltpu.make_async_copy(
            seq.at[pl.ds(r_, 8),
                   pl.ds(pl.multiple_of(CCH * g_, 128), CCH)],
            bufs[kb], sem_in[kb]).start()

    def wait_in(kb):
        pltpu.make_async_copy(
            seq.at[pl.ds(0, 8), pl.ds(0, CCH)], bufs[kb],
            sem_in[kb]).wait()

    def start_out(kb, t):
        r_, g_ = rg(t)
        pltpu.make_async_copy(
            bufs[kb],
            out.at[pl.ds(r_, 8),
                   pl.ds(pl.multiple_of(CCH * g_, 128), CCH)],
            sem_out[kb]).start()

    def wait_out(kb):
        pltpu.make_async_copy(
            bufs[kb],
            out.at[pl.ds(0, 8), pl.ds(0, CCH)], sem_out[kb]).wait()

    def apply_embed(kb, t):
        r_, g_ = rg(t)
        iv = idxv[pl.ds(r_, L)]
        o_lo = OCH * g_
        for r in range(8):
            o_b = iv[r]
            o_loc = o_b - o_lo
            hit = jnp.logical_and(o_b >= o_lo, o_b < o_lo + OCH)

            c0 = OBS * o_loc

            @pl.when(hit)
            def _():
                for j in range(OBS // L):
                    bufs[kb][r, pl.ds(c0 + L * j, L)] = (
                        bufs[kb][r, pl.ds(c0 + L * j, L)] + evecs[j])

    start_in(0, 0)
    start_in(1, 1)

    def body(i, carry):
        for kb in (0, 1):
            t = 2 * i + kb
            wait_in(kb)
            apply_embed(kb, t)
            start_out(kb, t)
            wait_out(kb)

            @pl.when(t + 2 < NCHUNK)
            def _():
                start_in(kb, t + 2)
        return carry

    lax.fori_loop(0, NCHUNK // 2, body, 0)

    skill_in.wait()
    pltpu.sync_copy(
        skbuf, out.at[pl.ds(b0, 128), pl.ds(SEQ_LEN * OBS, 128)])


@jax.jit
def kernel(seq, skill, obj_idx, obj_embed):
    obj_idx = obj_idx.astype(jnp.int32)
    seq = seq.reshape(BATCH, SEQ_LEN * OBS)
    mesh = plsc.VectorSubcoreMesh(core_axis_name="c", subcore_axis_name="s")
    out = pl.kernel(
        _sc_kernel,
        out_type=jax.ShapeDtypeStruct((BATCH, OUT_COLS), jnp.float32),
        mesh=mesh,
        scratch_types=[
            pltpu.VMEM((BATCH + L,), jnp.int32),
            pltpu.VMEM((OBS,), jnp.float32),
            pltpu.VMEM((128, 128), jnp.float32),
            pltpu.VMEM((8, CCH), jnp.float32),
            pltpu.VMEM((8, CCH), jnp.float32),
            pltpu.SemaphoreType.DMA,
            pltpu.SemaphoreType.DMA,
            pltpu.SemaphoreType.DMA,
            pltpu.SemaphoreType.DMA,
            pltpu.SemaphoreType.DMA,
        ],
    )(seq, skill, obj_idx, obj_embed)
    return out
